# Initial kernel scaffold; baseline (speedup 1.0000x reference)
import jax
import jax.numpy as jnp
from jax import lax
from jax.experimental import pallas as pl
from jax.experimental.pallas import tpu as pltpu
from jax.experimental.pallas import tpu_sc as plsc

RANK = 32
NNZ = 1048576

NUM_WORKERS = 32
LANES = 16
CHUNK = 512
GATHER = 128
N_PER_W = NNZ // NUM_WORKERS
N_ITERS = N_PER_W // CHUNK
N_GATHERS = CHUNK // GATHER


def _dot_kernel(x0_hbm, x1_hbm, a_hbm, b_hbm, out_hbm,
                idx0_v, idx1_v, rows_a, rows_b, out_v, sem_a, sem_b):
  nc = 2
  wid = lax.axis_index("s") * nc + lax.axis_index("c")
  lane_iota = lax.iota(jnp.int32, LANES)

  def body(it, _):
    base = wid * N_PER_W + it * CHUNK
    row = base // GATHER

    pltpu.sync_copy(x0_hbm.at[pl.ds(row, N_GATHERS)], idx0_v)
    pltpu.sync_copy(x1_hbm.at[pl.ds(row, N_GATHERS)], idx1_v)

    for j in range(N_GATHERS):
      pltpu.async_copy(a_hbm.at[idx0_v.at[j]],
                       rows_a.at[pl.ds(j * GATHER, GATHER)], sem_a)
      pltpu.async_copy(b_hbm.at[idx1_v.at[j]],
                       rows_b.at[pl.ds(j * GATHER, GATHER)], sem_b)
    for j in range(N_GATHERS):
      pltpu.make_async_copy(a_hbm.at[idx0_v.at[j]],
                            rows_a.at[pl.ds(j * GATHER, GATHER)], sem_a).wait()
      pltpu.make_async_copy(b_hbm.at[idx1_v.at[j]],
                            rows_b.at[pl.ds(j * GATHER, GATHER)], sem_b).wait()

    def blk_body(blk, _):
      rows16 = blk * LANES + lane_iota
      acc = jnp.zeros((LANES,), jnp.float32)
      for k in range(RANK):
        col = jnp.full((LANES,), k, jnp.int32)
        va = plsc.load_gather(rows_a, [rows16, col])
        vb = plsc.load_gather(rows_b, [rows16, col])
        acc = acc + va * vb
      out_v[pl.ds(blk * LANES, LANES)] = acc
      return 0

    lax.fori_loop(0, CHUNK // LANES, blk_body, 0)
    pltpu.sync_copy(out_v, out_hbm.at[pl.ds(base, CHUNK)])
    return 0

  lax.fori_loop(0, N_ITERS, body, 0)


def kernel(x, A, B):
  x0 = x[0].astype(jnp.int32).reshape(NNZ // GATHER, GATHER)
  x1 = x[1].astype(jnp.int32).reshape(NNZ // GATHER, GATHER)

  mesh = plsc.VectorSubcoreMesh(core_axis_name="c", subcore_axis_name="s")
  run = pl.kernel(
      _dot_kernel,
      out_type=jax.ShapeDtypeStruct((NNZ,), jnp.float32),
      mesh=mesh,
      scratch_types=[
          pltpu.VMEM((N_GATHERS, GATHER), jnp.int32),
          pltpu.VMEM((N_GATHERS, GATHER), jnp.int32),
          pltpu.VMEM((CHUNK, RANK), jnp.float32),
          pltpu.VMEM((CHUNK, RANK), jnp.float32),
          pltpu.VMEM((CHUNK,), jnp.float32),
          pltpu.SemaphoreType.DMA,
          pltpu.SemaphoreType.DMA,
      ],
  )
  return run(x0, x1, A, B)


# speedup vs baseline: 2.4097x; 2.4097x over previous
import jax
import jax.numpy as jnp
from jax import lax
from jax.experimental import pallas as pl
from jax.experimental.pallas import tpu as pltpu
from jax.experimental.pallas import tpu_sc as plsc

RANK = 32
NNZ = 1048576

NUM_WORKERS = 32
LANES = 16
CHUNK = 512
GATHER = 128
N_PER_W = NNZ // NUM_WORKERS
N_ITERS = N_PER_W // CHUNK
N_GATHERS = CHUNK // GATHER
NBUF = 2

_DNUMS = lax.GatherDimensionNumbers(
    offset_dims=(), collapsed_slice_dims=(0,), start_index_map=(0,))


def _shuf(w, idx):
  return lax.gather(w, idx[:, None], _DNUMS, (1,),
                    mode=lax.GatherScatterMode.PROMISE_IN_BOUNDS)


def _dot_kernel(x0_hbm, x1_hbm, a_hbm, b_hbm, out_hbm,
                idx0_v, idx1_v, rows_a, rows_b, out_v, sem_g, sem_o):
  nc = 2
  wid = lax.axis_index("s") * nc + lax.axis_index("c")
  lane_iota = lax.iota(jnp.int32, LANES)
  shuf_idx = [jnp.bitwise_xor(lane_iota, sh) for sh in (8, 4, 2, 1)]

  def fire(it, buf):
    base = wid * N_PER_W + it * CHUNK
    row = pl.multiple_of(base // GATHER, N_GATHERS)
    pltpu.sync_copy(x0_hbm.at[pl.ds(row, N_GATHERS)], idx0_v.at[buf])
    pltpu.sync_copy(x1_hbm.at[pl.ds(row, N_GATHERS)], idx1_v.at[buf])
    for j in range(N_GATHERS):
      pltpu.async_copy(a_hbm.at[idx0_v.at[buf, j]],
                       rows_a.at[buf, pl.ds(j * GATHER, GATHER)], sem_g.at[buf])
      pltpu.async_copy(b_hbm.at[idx1_v.at[buf, j]],
                       rows_b.at[buf, pl.ds(j * GATHER, GATHER)], sem_g.at[buf])

  def drain(buf):
    for j in range(N_GATHERS):
      pltpu.make_async_copy(a_hbm.at[idx0_v.at[buf, j]],
                            rows_a.at[buf, pl.ds(j * GATHER, GATHER)],
                            sem_g.at[buf]).wait()
      pltpu.make_async_copy(b_hbm.at[idx1_v.at[buf, j]],
                            rows_b.at[buf, pl.ds(j * GATHER, GATHER)],
                            sem_g.at[buf]).wait()

  for b in range(NBUF):
    fire(b, b)

  def body(it, _):
    buf = lax.rem(it, NBUF)
    drain(buf)

    @pl.when(it >= NBUF)
    def _():
      prev_base = wid * N_PER_W + (it - NBUF) * CHUNK
      pltpu.make_async_copy(out_v.at[buf], out_hbm.at[pl.ds(prev_base, CHUNK)],
                            sem_o.at[buf]).wait()

    def blk_body(blk, _):
      r0 = blk * LANES
      o = jnp.zeros((LANES,), jnp.float32)
      for r in range(LANES):
        row_i = r0 + r
        a_lo = rows_a[buf, row_i, pl.ds(0, LANES)]
        a_hi = rows_a[buf, row_i, pl.ds(LANES, LANES)]
        b_lo = rows_b[buf, row_i, pl.ds(0, LANES)]
        b_hi = rows_b[buf, row_i, pl.ds(LANES, LANES)]
        w = a_lo * b_lo + a_hi * b_hi
        for si in shuf_idx:
          w = w + _shuf(w, si)
        o = jnp.where(lane_iota == r, w, o)
      out_v[buf, pl.ds(r0, LANES)] = o
      return 0

    lax.fori_loop(0, CHUNK // LANES, blk_body, 0)
    base = wid * N_PER_W + it * CHUNK
    pltpu.async_copy(out_v.at[buf], out_hbm.at[pl.ds(base, CHUNK)], sem_o.at[buf])

    @pl.when(it + NBUF < N_ITERS)
    def _():
      fire(it + NBUF, buf)
    return 0

  lax.fori_loop(0, N_ITERS, body, 0)
  for b in range(NBUF):
    it = N_ITERS - NBUF + b
    buf = it % NBUF
    base = wid * N_PER_W + it * CHUNK
    pltpu.make_async_copy(out_v.at[buf], out_hbm.at[pl.ds(base, CHUNK)],
                          sem_o.at[buf]).wait()


def kernel(x, A, B):
  x0 = x[0].astype(jnp.int32).reshape(NNZ // GATHER, GATHER)
  x1 = x[1].astype(jnp.int32).reshape(NNZ // GATHER, GATHER)

  mesh = plsc.VectorSubcoreMesh(core_axis_name="c", subcore_axis_name="s")
  run = pl.kernel(
      _dot_kernel,
      out_type=jax.ShapeDtypeStruct((NNZ,), jnp.float32),
      mesh=mesh,
      scratch_types=[
          pltpu.VMEM((NBUF, N_GATHERS, GATHER), jnp.int32),
          pltpu.VMEM((NBUF, N_GATHERS, GATHER), jnp.int32),
          pltpu.VMEM((NBUF, CHUNK, RANK), jnp.float32),
          pltpu.VMEM((NBUF, CHUNK, RANK), jnp.float32),
          pltpu.VMEM((NBUF, CHUNK), jnp.float32),
          pltpu.SemaphoreType.DMA((NBUF,)),
          pltpu.SemaphoreType.DMA((NBUF,)),
      ],
      compiler_params=pltpu.CompilerParams(use_tc_tiling_on_sc=False),
  )
  return run(x0, x1, A, B)


# speedup vs baseline: 2.4129x; 1.0013x over previous
import jax
import jax.numpy as jnp
from jax import lax
from jax.experimental import pallas as pl
from jax.experimental.pallas import tpu as pltpu
from jax.experimental.pallas import tpu_sc as plsc

RANK = 32
NNZ = 1048576

NUM_WORKERS = 32
LANES = 16
CHUNK = 512
GATHER = 128
N_PER_W = NNZ // NUM_WORKERS
N_ITERS = N_PER_W // CHUNK
N_GATHERS = CHUNK // GATHER
NBUF = 2

_DNUMS = lax.GatherDimensionNumbers(
    offset_dims=(), collapsed_slice_dims=(0,), start_index_map=(0,))


def _shuf(w, idx):
  return lax.gather(w, idx[:, None], _DNUMS, (1,),
                    mode=lax.GatherScatterMode.PROMISE_IN_BOUNDS)


def _dot_kernel(x_hbm, a_hbm, b_hbm, out_hbm,
                idx0_v, idx1_v, rows_a, rows_b, out_v, sem_g, sem_o):
  nc = 2
  wid = lax.axis_index("s") * nc + lax.axis_index("c")
  lane_iota = lax.iota(jnp.int32, LANES)
  shuf_idx = [jnp.bitwise_xor(lane_iota, sh) for sh in (8, 4, 2, 1)]

  def fire(it, buf):
    base = pl.multiple_of(wid * N_PER_W + it * CHUNK, CHUNK)
    pltpu.sync_copy(x_hbm.at[0, pl.ds(base, CHUNK)], idx0_v.at[buf])
    pltpu.sync_copy(x_hbm.at[1, pl.ds(base, CHUNK)], idx1_v.at[buf])
    for j in range(N_GATHERS):
      pltpu.async_copy(a_hbm.at[idx0_v.at[buf, pl.ds(j * GATHER, GATHER)]],
                       rows_a.at[buf, pl.ds(j * GATHER, GATHER)], sem_g.at[buf])
      pltpu.async_copy(b_hbm.at[idx1_v.at[buf, pl.ds(j * GATHER, GATHER)]],
                       rows_b.at[buf, pl.ds(j * GATHER, GATHER)], sem_g.at[buf])

  def drain(buf):
    for j in range(N_GATHERS):
      pltpu.make_async_copy(a_hbm.at[idx0_v.at[buf, pl.ds(j * GATHER, GATHER)]],
                            rows_a.at[buf, pl.ds(j * GATHER, GATHER)],
                            sem_g.at[buf]).wait()
      pltpu.make_async_copy(b_hbm.at[idx1_v.at[buf, pl.ds(j * GATHER, GATHER)]],
                            rows_b.at[buf, pl.ds(j * GATHER, GATHER)],
                            sem_g.at[buf]).wait()

  for b in range(NBUF):
    fire(b, b)

  def body(it, _):
    buf = lax.rem(it, NBUF)
    drain(buf)

    @pl.when(it >= NBUF)
    def _():
      prev_base = wid * N_PER_W + (it - NBUF) * CHUNK
      pltpu.make_async_copy(out_v.at[buf], out_hbm.at[pl.ds(prev_base, CHUNK)],
                            sem_o.at[buf]).wait()

    def blk_body(blk, _):
      r0 = blk * LANES
      o = jnp.zeros((LANES,), jnp.float32)
      for r in range(LANES):
        row_i = r0 + r
        a_lo = rows_a[buf, row_i, pl.ds(0, LANES)]
        a_hi = rows_a[buf, row_i, pl.ds(LANES, LANES)]
        b_lo = rows_b[buf, row_i, pl.ds(0, LANES)]
        b_hi = rows_b[buf, row_i, pl.ds(LANES, LANES)]
        w = a_lo * b_lo + a_hi * b_hi
        for si in shuf_idx:
          w = w + _shuf(w, si)
        o = jnp.where(lane_iota == r, w, o)
      out_v[buf, pl.ds(r0, LANES)] = o
      return 0

    lax.fori_loop(0, CHUNK // LANES, blk_body, 0)
    base = wid * N_PER_W + it * CHUNK
    pltpu.async_copy(out_v.at[buf], out_hbm.at[pl.ds(base, CHUNK)], sem_o.at[buf])

    @pl.when(it + NBUF < N_ITERS)
    def _():
      fire(it + NBUF, buf)
    return 0

  lax.fori_loop(0, N_ITERS, body, 0)
  for b in range(NBUF):
    it = N_ITERS - NBUF + b
    buf = it % NBUF
    base = wid * N_PER_W + it * CHUNK
    pltpu.make_async_copy(out_v.at[buf], out_hbm.at[pl.ds(base, CHUNK)],
                          sem_o.at[buf]).wait()


def kernel(x, A, B):
  if x.dtype != jnp.int32:
    x = x.astype(jnp.int32)

  mesh = plsc.VectorSubcoreMesh(core_axis_name="c", subcore_axis_name="s")
  run = pl.kernel(
      _dot_kernel,
      out_type=jax.ShapeDtypeStruct((NNZ,), jnp.float32),
      mesh=mesh,
      scratch_types=[
          pltpu.VMEM((NBUF, CHUNK), jnp.int32),
          pltpu.VMEM((NBUF, CHUNK), jnp.int32),
          pltpu.VMEM((NBUF, CHUNK, RANK), jnp.float32),
          pltpu.VMEM((NBUF, CHUNK, RANK), jnp.float32),
          pltpu.VMEM((NBUF, CHUNK), jnp.float32),
          pltpu.SemaphoreType.DMA((NBUF,)),
          pltpu.SemaphoreType.DMA((NBUF,)),
      ],
      compiler_params=pltpu.CompilerParams(use_tc_tiling_on_sc=False),
  )
  return run(x, A, B)
